# baseline (speedup 1.0000x reference)
import functools

import jax
import jax.numpy as jnp
from jax import lax
from jax.experimental import pallas as pl
from jax.experimental.pallas import tpu as pltpu
from jax.experimental.pallas import tpu_sc as plsc

B = 16384
D = 64
K = 20
LANES = 16

NC = 2
NS = 16
NW = NC * NS

BPW = B // NW
CB = 32
NCHUNK = BPW // CB
NGROUP = CB // LANES
NIDX_ROWS = CB * K // 128
DC = 8


def _log_sigmoid(x):
    t = jnp.exp(-jnp.abs(x))
    z = t / (t + 2.0)
    z2 = z * z
    p = 1.0 + z2 * (1.0 / 3.0 + z2 * (1.0 / 5.0 + z2 * (1.0 / 7.0 + z2 * (1.0 / 9.0))))
    return jnp.minimum(x, 0.0) - 2.0 * z * p


def _sc_body(center_ref, pos_ref, neg_ref, ein_ref, eout_ref, out_ref,
             cidx0, cidx1, pidx0, pidx1, nidx0, nidx1,
             c0, c1, p0, p1, n0, n1, lacc, sem0, sem1):
    wid = lax.axis_index("s") * NC + lax.axis_index("c")
    cidx = (cidx0, cidx1)
    pidx = (pidx0, pidx1)
    nidx = (nidx0, nidx1)
    crows = (c0, c1)
    prows = (p0, p1)
    nrows = (n0, n1)
    sems = (sem0, sem1)

    iota16 = lax.iota(jnp.int32, LANES)
    perms = [iota16 ^ sh for sh in (1, 2, 4, 8)]

    def hsum(t):
        for p in perms:
            t = t + jnp.take(t, p)
        return t

    lacc[...] = jnp.zeros((LANES,), jnp.float32)

    def issue(g, s):
        cbase = wid * BPW + g * CB
        pltpu.sync_copy(center_ref.at[pl.ds(cbase, CB)], cidx[s])
        pltpu.sync_copy(pos_ref.at[pl.ds(cbase, CB)], pidx[s])
        pltpu.sync_copy(neg_ref.at[pl.ds(cbase * K, CB * K)], nidx[s])
        pltpu.async_copy(ein_ref.at[cidx[s]], crows[s], sems[s])
        pltpu.async_copy(eout_ref.at[pidx[s]], prows[s], sems[s])
        for j in range(NIDX_ROWS):
            pltpu.async_copy(
                eout_ref.at[nidx[s].at[pl.ds(j * 128, 128)]],
                nrows[s].at[pl.ds(j * 128, 128)], sems[s])

    def drain(s):
        pltpu.make_async_copy(ein_ref.at[pl.ds(0, CB)], crows[s], sems[s]).wait()
        pltpu.make_async_copy(eout_ref.at[pl.ds(0, CB)], prows[s], sems[s]).wait()
        pltpu.make_async_copy(eout_ref.at[pl.ds(0, CB * K)], nrows[s], sems[s]).wait()

    def compute(s):
        cb, pb, nb = crows[s], prows[s], nrows[s]

        def group_body(g2, carry):
            def b_body(bl, slots):
                b = g2 * LANES + bl
                mask = iota16 == bl

                cq = [cb[b, pl.ds(m * LANES, LANES)] for m in range(D // LANES)]

                def dot_row(ref, r):
                    t = cq[0] * ref[r, pl.ds(0, LANES)]
                    for m in range(1, D // LANES):
                        t = t + cq[m] * ref[r, pl.ds(m * LANES, LANES)]
                    return hsum(t)

                news = [None] * (K + 1)
                news[0] = jnp.where(mask, dot_row(pb, b), slots[0])
                nbase = b * K
                for k in range(K):
                    news[1 + k] = jnp.where(mask, dot_row(nb, nbase + k),
                                            slots[1 + k])
                return tuple(news)

            zero = jnp.zeros((LANES,), jnp.float32)
            slots = lax.fori_loop(0, LANES, b_body, (zero,) * (K + 1))
            tot = _log_sigmoid(slots[0])
            for k in range(K):
                tot = tot + _log_sigmoid(-slots[1 + k])
            lacc[...] = lacc[...] + tot
            return carry

        lax.fori_loop(0, NGROUP, group_body, 0)

    issue(0, 0)
    issue(1, 1)

    def t_body(i, carry):
        for s in (0, 1):
            g = i * 2 + s
            drain(s)
            compute(s)
            ng = g + 2

            @pl.when(ng < NCHUNK)
            def _():
                issue(ng, s)
        return carry

    lax.fori_loop(0, NCHUNK // 2, t_body, 0)
    pltpu.sync_copy(lacc, out_ref.at[wid])


@functools.cache
def _build_sc_kernel():
    mesh = plsc.VectorSubcoreMesh(
        core_axis_name="c", subcore_axis_name="s",
        num_cores=NC, num_subcores=NS)
    return pl.kernel(
        _sc_body,
        out_type=jax.ShapeDtypeStruct((NW, LANES), jnp.float32),
        mesh=mesh,
        compiler_params=pltpu.CompilerParams(use_tc_tiling_on_sc=False),
        scratch_types=[
            pltpu.VMEM((CB,), jnp.int32), pltpu.VMEM((CB,), jnp.int32),
            pltpu.VMEM((CB,), jnp.int32), pltpu.VMEM((CB,), jnp.int32),
            pltpu.VMEM((CB * K,), jnp.int32),
            pltpu.VMEM((CB * K,), jnp.int32),
            pltpu.VMEM((CB, D), jnp.float32), pltpu.VMEM((CB, D), jnp.float32),
            pltpu.VMEM((CB, D), jnp.float32), pltpu.VMEM((CB, D), jnp.float32),
            pltpu.VMEM((CB * K, D), jnp.float32),
            pltpu.VMEM((CB * K, D), jnp.float32),
            pltpu.VMEM((LANES,), jnp.float32),
            pltpu.SemaphoreType.DMA, pltpu.SemaphoreType.DMA,
        ],
    )


def kernel(center_ids, pos_ids, neg_ids, emb_in_weight, emb_out_weight):
    neg_flat = neg_ids.astype(jnp.int32).reshape(B * K)
    partials = _build_sc_kernel()(center_ids.astype(jnp.int32),
                                  pos_ids.astype(jnp.int32),
                                  neg_flat, emb_in_weight, emb_out_weight)
    return -jnp.sum(partials) / B
